# baseline (speedup 1.0000x reference)
import jax
import jax.numpy as jnp
import numpy as np
from jax.experimental import pallas as pl
from jax.experimental.pallas import tpu as pltpu
from jax.sharding import Mesh, PartitionSpec as P

B, QD, AD, FD = 4096, 4096, 4096, 16384
BLK = 256
NE = 8


def _top2_softmax(l):
    col = jax.lax.broadcasted_iota(jnp.int32, l.shape, 1)
    m1 = jnp.max(l, axis=-1, keepdims=True)
    i1 = jnp.min(jnp.where(l == m1, col, NE), axis=-1, keepdims=True)
    l2 = jnp.where(col == i1, -jnp.inf, l)
    m2 = jnp.max(l2, axis=-1, keepdims=True)
    i2 = jnp.min(jnp.where(l2 == m2, col, NE), axis=-1, keepdims=True)
    s = jnp.exp(m2 - m1)
    w1 = 1.0 / (1.0 + s)
    w2 = 1.0 - w1
    return jnp.where(col == i1, w1, 0.0) + jnp.where(col == i2, w2, 0.0)


def _body(x_ref, wa_ref, ba_ref, wf_ref, bf_ref,
          eav_ref, eat_ref, eag_ref, efv_ref, eft_ref, efg_ref,
          la_ref, lf_ref, va_ref, vf_ref):
    x = x_ref[...]
    la = jax.lax.dot_general(
        x, wa_ref[...], (((1,), (1,)), ((), ())),
        preferred_element_type=jnp.float32) + ba_ref[...]
    lf = jax.lax.dot_general(
        x, wf_ref[...], (((1,), (1,)), ((), ())),
        preferred_element_type=jnp.float32) + bf_ref[...]
    la_ref[...] = la
    lf_ref[...] = lf
    ones = jnp.ones((x.shape[0], 1), jnp.float32)
    wa = jnp.concatenate([_top2_softmax(la), ones], axis=1)
    wf = jnp.concatenate([_top2_softmax(lf), ones], axis=1)
    ea = jnp.concatenate([eav_ref[...], eat_ref[...], eag_ref[...]], axis=0)
    ef = jnp.concatenate([efv_ref[...], eft_ref[...], efg_ref[...]], axis=0)
    va_ref[...] = jax.lax.dot_general(
        wa, ea, (((1,), (0,)), ((), ())),
        preferred_element_type=jnp.float32)
    vf_ref[...] = jax.lax.dot_general(
        wf, ef, (((1,), (0,)), ((), ())),
        preferred_element_type=jnp.float32)


def _full(shape):
    return pl.BlockSpec(shape, lambda i: tuple(0 for _ in shape))


def _run_shard(x, wa, ba, wf, bf, eav, eat, eag, efv, eft, efg):
    nb = x.shape[0]
    grid = (nb // BLK,)
    la, lf, va, vf = pl.pallas_call(
        _body,
        grid=grid,
        in_specs=[
            pl.BlockSpec((BLK, QD), lambda i: (i, 0)),
            _full((NE, QD)), _full((1, NE)),
            _full((NE, QD)), _full((1, NE)),
            _full((4, AD)), _full((4, AD)), _full((1, AD)),
            _full((4, FD)), _full((4, FD)), _full((1, FD)),
        ],
        out_specs=[
            pl.BlockSpec((BLK, NE), lambda i: (i, 0)),
            pl.BlockSpec((BLK, NE), lambda i: (i, 0)),
            pl.BlockSpec((BLK, AD), lambda i: (i, 0)),
            pl.BlockSpec((BLK, FD), lambda i: (i, 0)),
        ],
        out_shape=[
            jax.ShapeDtypeStruct((nb, NE), jnp.float32),
            jax.ShapeDtypeStruct((nb, NE), jnp.float32),
            jax.ShapeDtypeStruct((nb, AD), jnp.float32),
            jax.ShapeDtypeStruct((nb, FD), jnp.float32),
        ],
        compiler_params=pltpu.CompilerParams(
            dimension_semantics=("arbitrary",),
        ),
    )(x, wa, ba, wf, bf, eav, eat, eag, efv, eft, efg)
    return la, lf, va, vf


@jax.jit
def kernel(query_features, W_attn, b_attn, W_ffn, b_ffn,
           E_attn_vis, E_attn_text, E_attn_general,
           E_ffn_vis, E_ffn_text, E_ffn_general):
    devs = jax.devices()
    ndev = 1
    for n in (2, 4, 8):
        if len(devs) >= n and (B // n) % BLK == 0:
            ndev = n
    mesh = Mesh(np.array(devs[:ndev]), ("d",))
    rep = P(None, None)
    f = jax.shard_map(
        _run_shard, mesh=mesh,
        in_specs=(P("d", None),) + (rep,) * 10,
        out_specs=(P("d", None), P("d", None), P("d", None), P("d", None)),
        check_vma=False,
    )
    la, lf, va, vf = f(
        query_features, W_attn, b_attn[None, :], W_ffn, b_ffn[None, :],
        E_attn_vis, E_attn_text, E_attn_general,
        E_ffn_vis, E_ffn_text, E_ffn_general)
    return (va, vf, la, lf)


# speedup vs baseline: 1.2399x; 1.2399x over previous
import jax
import jax.numpy as jnp
import numpy as np
from jax.experimental import pallas as pl
from jax.experimental.pallas import tpu as pltpu
from jax.sharding import Mesh, PartitionSpec as P

B, QD, AD, FD = 4096, 4096, 4096, 16384
BLK = 256
NE = 8


def _top2_softmax(l):
    col = jax.lax.broadcasted_iota(jnp.int32, l.shape, 1)
    m1 = jnp.max(l, axis=-1, keepdims=True)
    i1 = jnp.min(jnp.where(l == m1, col, NE), axis=-1, keepdims=True)
    l2 = jnp.where(col == i1, -jnp.inf, l)
    m2 = jnp.max(l2, axis=-1, keepdims=True)
    i2 = jnp.min(jnp.where(l2 == m2, col, NE), axis=-1, keepdims=True)
    s = jnp.exp(m2 - m1)
    w1 = 1.0 / (1.0 + s)
    w2 = 1.0 - w1
    return jnp.where(col == i1, w1, 0.0) + jnp.where(col == i2, w2, 0.0)


def _body(x_ref, wa_ref, ba_ref, wf_ref, bf_ref,
          eav_ref, eat_ref, eag_ref, efv_ref, eft_ref, efg_ref,
          la_ref, lf_ref, va_ref, vf_ref):
    x = x_ref[...]
    la = jax.lax.dot_general(
        x, wa_ref[...], (((1,), (1,)), ((), ())),
        preferred_element_type=jnp.float32) + ba_ref[...]
    lf = jax.lax.dot_general(
        x, wf_ref[...], (((1,), (1,)), ((), ())),
        preferred_element_type=jnp.float32) + bf_ref[...]
    la_ref[...] = la
    lf_ref[...] = lf
    ones = jnp.ones((x.shape[0], 1), jnp.float32)
    wa = jnp.concatenate([_top2_softmax(la), ones], axis=1)
    wf = jnp.concatenate([_top2_softmax(lf), ones], axis=1)
    ea = jnp.concatenate([eav_ref[...], eat_ref[...], eag_ref[...]], axis=0)
    ef = jnp.concatenate([efv_ref[...], eft_ref[...], efg_ref[...]], axis=0)
    va_ref[...] = jax.lax.dot_general(
        wa, ea, (((1,), (0,)), ((), ())),
        preferred_element_type=jnp.float32)
    vf_ref[...] = jax.lax.dot_general(
        wf, ef, (((1,), (0,)), ((), ())),
        preferred_element_type=jnp.float32)


def _full(shape):
    return pl.BlockSpec(shape, lambda i: tuple(0 for _ in shape))


def _run_shard(x, wa, ba, wf, bf, eav, eat, eag, efv, eft, efg):
    nb = x.shape[0]
    grid = (nb // BLK,)
    la, lf, va, vf = pl.pallas_call(
        _body,
        grid=grid,
        in_specs=[
            pl.BlockSpec((BLK, QD), lambda i: (i, 0)),
            _full((NE, QD)), _full((1, NE)),
            _full((NE, QD)), _full((1, NE)),
            _full((4, AD)), _full((4, AD)), _full((1, AD)),
            _full((4, FD)), _full((4, FD)), _full((1, FD)),
        ],
        out_specs=[
            pl.BlockSpec((BLK, NE), lambda i: (i, 0)),
            pl.BlockSpec((BLK, NE), lambda i: (i, 0)),
            pl.BlockSpec((BLK, AD), lambda i: (i, 0)),
            pl.BlockSpec((BLK, FD), lambda i: (i, 0)),
        ],
        out_shape=[
            jax.ShapeDtypeStruct((nb, NE), jnp.float32),
            jax.ShapeDtypeStruct((nb, NE), jnp.float32),
            jax.ShapeDtypeStruct((nb, AD), jnp.float32),
            jax.ShapeDtypeStruct((nb, FD), jnp.float32),
        ],
        compiler_params=pltpu.CompilerParams(
            dimension_semantics=("arbitrary",),
        ),
    )(x, wa, ba, wf, bf, eav, eat, eag, efv, eft, efg)
    return la, lf, va, vf


@jax.jit
def kernel(query_features, W_attn, b_attn, W_ffn, b_ffn,
           E_attn_vis, E_attn_text, E_attn_general,
           E_ffn_vis, E_ffn_text, E_ffn_general):
    devs = jax.devices()
    ndev = 1
    for n in ():
        if len(devs) >= n and (B // n) % BLK == 0:
            ndev = n
    mesh = Mesh(np.array(devs[:ndev]), ("d",))
    rep = P(None, None)
    f = jax.shard_map(
        _run_shard, mesh=mesh,
        in_specs=(P("d", None),) + (rep,) * 10,
        out_specs=(P("d", None), P("d", None), P("d", None), P("d", None)),
        check_vma=False,
    )
    la, lf, va, vf = f(
        query_features, W_attn, b_attn[None, :], W_ffn, b_ffn[None, :],
        E_attn_vis, E_attn_text, E_attn_general,
        E_ffn_vis, E_ffn_text, E_ffn_general)
    return (va, vf, la, lf)
